# Initial kernel scaffold; baseline (speedup 1.0000x reference)
import jax
import jax.numpy as jnp
from jax.experimental import pallas as pl


def kernel(t, f0, index, v0, epsilon_v, epsilon_r):
    raise NotImplementedError("write your pallas kernel here")


# speedup vs baseline: 1.5021x; 1.5021x over previous
import functools
import math

import jax
import jax.numpy as jnp
from jax import lax
from jax.experimental import pallas as pl
from jax.experimental.pallas import tpu as pltpu
from jax.experimental.pallas import tpu_sc as plsc

N = 3200000
S = 100000
EPS = 1e-05
VEL = 1.0 / (2.0 * math.pi)
NC = 2
NS = 16
NW = NC * NS

f32 = jnp.float32
i32 = jnp.int32


def _sqrt(x):
    i = lax.bitcast_convert_type(x, i32)
    y = lax.bitcast_convert_type((i >> 1) + 0x1FBD1DF5, f32)
    y = 0.5 * (y + x / y)
    y = 0.5 * (y + x / y)
    y = 0.5 * (y + x / y)
    return y


def _wrap(x):
    m = lax.rem(x + 0.5, 1.0)
    m = jnp.where(m < 0.0, m + 1.0, m)
    return m - 0.5


def _make(n, s, t_tile, interpret=False):
    chunk = n // NW
    nt = chunk // t_tile
    flat = 3 * t_tile
    s8 = s * 8
    eb = s8 // NS
    assert n == NW * chunk and chunk == nt * t_tile
    assert t_tile % 16 == 0 and s8 % NS == 0 and eb % 8 == 0
    mesh = plsc.VectorSubcoreMesh(
        core_axis_name="c", subcore_axis_name="s",
        num_cores=NC, num_subcores=NS)
    cparams = pltpu.CompilerParams(use_tc_tiling_on_sc=False)

    def _wid():
        return lax.axis_index("s") * NC + lax.axis_index("c")

    def _zero_table(z_hbm, table):
        sid = lax.axis_index("s")
        pltpu.sync_copy(z_hbm.at[pl.ds(pl.multiple_of(sid * eb, 8), eb)],
                        table.at[pl.ds(pl.multiple_of(sid * eb, 8), eb)])

    def _export_table(table, out):
        sid = lax.axis_index("s")
        cid = lax.axis_index("c")
        pltpu.sync_copy(table.at[pl.ds(pl.multiple_of(sid * eb, 8), eb)],
                        out.at[cid, pl.ds(pl.multiple_of(sid * eb, 8), eb)])

    L = 128
    rb = 200
    nr = n // L
    assert n % L == 0 and nr % rb == 0

    def _tc0_body(ix, tt, sl_o, sle_o, slc_o, t3_o):
        base = ix[...] * 8
        rep = jnp.reshape(jnp.broadcast_to(base[..., None], (rb, L, 3)),
                          (rb, L * 3))
        c3 = lax.broadcasted_iota(i32, (rb, L * 3), 1) % 3
        sl_o[...] = rep + c3
        sle_o[...] = rep + c3 + 3
        slc_o[...] = base + 6
        t3_o[...] = jnp.reshape(
            jnp.broadcast_to(tt[...][..., None], (rb, L, 3)), (rb, L * 3))

    tc0 = pl.pallas_call(
        _tc0_body,
        out_shape=(jax.ShapeDtypeStruct((nr, L * 3), i32),
                   jax.ShapeDtypeStruct((nr, L * 3), i32),
                   jax.ShapeDtypeStruct((nr, L), i32),
                   jax.ShapeDtypeStruct((nr, L * 3), f32)),
        grid=(nr // rb,),
        in_specs=[pl.BlockSpec((rb, L), lambda i: (i, 0)),
                  pl.BlockSpec((rb, L), lambda i: (i, 0))],
        out_specs=(pl.BlockSpec((rb, L * 3), lambda i: (i, 0)),
                   pl.BlockSpec((rb, L * 3), lambda i: (i, 0)),
                   pl.BlockSpec((rb, L), lambda i: (i, 0)),
                   pl.BlockSpec((rb, L * 3), lambda i: (i, 0))),
        interpret=interpret,
    )

    @functools.partial(
        pl.kernel,
        out_type=jax.ShapeDtypeStruct((NC, s8), f32),
        mesh=mesh, interpret=interpret, compiler_params=cparams,
        scratch_types=[
            pltpu.VMEM((flat,), f32),
            pltpu.VMEM((flat,), f32),
            pltpu.VMEM((flat,), i32),
            pltpu.VMEM((flat,), i32),
            pltpu.VMEM((t_tile,), i32),
            pltpu.VMEM((t_tile,), f32),
            pltpu.VMEM_SHARED((s8,), f32),
        ],
    )
    def k1(ev_hbm, er_hbm, sl_hbm, sle_hbm, slc_hbm, z_hbm, out_hbm,
           evb, erb, slb, sleb, slcb, ones, table):
        w = _wid()
        _zero_table(z_hbm, table)

        def oinit(i, _):
            ones[pl.ds(i * 16, 16)] = jnp.ones((16,), f32)
            return 0
        lax.fori_loop(0, t_tile // 16, oinit, 0)
        plsc.subcore_barrier()

        def tile(k, _):
            row0 = w * chunk + k * t_tile
            f0 = pl.multiple_of(row0 * 3, 8)
            r0 = pl.multiple_of(row0, 8)
            pltpu.sync_copy(ev_hbm.at[pl.ds(f0, flat)], evb)
            pltpu.sync_copy(er_hbm.at[pl.ds(f0, flat)], erb)
            pltpu.sync_copy(sl_hbm.at[pl.ds(f0, flat)], slb)
            pltpu.sync_copy(sle_hbm.at[pl.ds(f0, flat)], sleb)
            pltpu.sync_copy(slc_hbm.at[pl.ds(r0, t_tile)], slcb)
            pltpu.sync_copy(evb, table.at[slb.at[pl.ds(0, flat)]], add=True)
            pltpu.sync_copy(erb, table.at[sleb.at[pl.ds(0, flat)]], add=True)
            pltpu.sync_copy(ones, table.at[slcb.at[pl.ds(0, t_tile)]],
                            add=True)
            return 0
        lax.fori_loop(0, nt, tile, 0)
        plsc.subcore_barrier()
        _export_table(table, out_hbm)

    bs = 2000
    assert s % bs == 0

    def _k1b_body(p0, p1, mv_o, mr_o):
        x = p0[0] + p1[0]
        cnt = jnp.maximum(x[:, 6:7], 1.0)
        col = lax.broadcasted_iota(i32, (bs, 8), 1)
        mv_o[...] = jnp.where(col < 3, x / cnt, 0.0)
        mr_o[...] = jnp.concatenate(
            [x[:, 3:6] / cnt, jnp.zeros((bs, 5), f32)], axis=1)

    k1b = pl.pallas_call(
        _k1b_body,
        out_shape=(jax.ShapeDtypeStruct((s, 8), f32),
                   jax.ShapeDtypeStruct((s, 8), f32)),
        grid=(s // bs,),
        in_specs=[pl.BlockSpec((1, bs, 8), lambda i: (0, i, 0)),
                  pl.BlockSpec((1, bs, 8), lambda i: (1, i, 0))],
        out_specs=(pl.BlockSpec((bs, 8), lambda i: (i, 0)),
                   pl.BlockSpec((bs, 8), lambda i: (i, 0))),
        interpret=interpret,
    )

    of = jax.ShapeDtypeStruct((3 * n,), f32)

    @functools.partial(
        pl.kernel,
        out_type=(of, of, of, of, of,
                  jax.ShapeDtypeStruct((NC, s8), f32)),
        mesh=mesh, interpret=interpret, compiler_params=cparams,
        scratch_types=[
            pltpu.VMEM((flat,), f32),
            pltpu.VMEM((flat,), f32),
            pltpu.VMEM((flat,), f32),
            pltpu.VMEM((flat,), f32),
            pltpu.VMEM((flat,), f32),
            pltpu.VMEM((flat,), i32),
            pltpu.VMEM((flat,), f32),
            pltpu.VMEM((flat,), f32),
            pltpu.VMEM((flat,), f32),
            pltpu.VMEM((flat,), f32),
            pltpu.VMEM((flat,), f32),
            pltpu.VMEM((flat,), f32),
            pltpu.VMEM((flat,), f32),
            pltpu.VMEM_SHARED((s8,), f32),
            pltpu.SemaphoreType.DMA,
            pltpu.SemaphoreType.DMA,
        ],
    )
    def k2(t3_hbm, f0_hbm, v0_hbm, ev_hbm, er_hbm, sl_hbm, mv_hbm, mr_hbm,
           z_hbm,
           vt_out, evc_out, erc_out, rt_out, fraw_out, fp_out,
           t3b, f0b, v0b, evb, erb, slb, mvb, mrb,
           vtb, evcb, ercb, rtb, frawb, ftable, sem1, sem2):
        w = _wid()
        _zero_table(z_hbm, ftable)
        plsc.subcore_barrier()

        def tile(k, _):
            row0 = w * chunk + k * t_tile
            f0o = pl.multiple_of(row0 * 3, 8)
            pltpu.sync_copy(sl_hbm.at[pl.ds(f0o, flat)], slb)
            pltpu.async_copy(mv_hbm.at[slb.at[pl.ds(0, flat)]], mvb, sem1)
            pltpu.async_copy(mr_hbm.at[slb.at[pl.ds(0, flat)]], mrb, sem2)
            pltpu.sync_copy(t3_hbm.at[pl.ds(f0o, flat)], t3b)
            pltpu.sync_copy(f0_hbm.at[pl.ds(f0o, flat)], f0b)
            pltpu.sync_copy(v0_hbm.at[pl.ds(f0o, flat)], v0b)
            pltpu.sync_copy(ev_hbm.at[pl.ds(f0o, flat)], evb)
            pltpu.sync_copy(er_hbm.at[pl.ds(f0o, flat)], erb)
            pltpu.make_async_copy(mv_hbm.at[slb.at[pl.ds(0, flat)]],
                                  mvb, sem1).wait()
            pltpu.make_async_copy(mr_hbm.at[slb.at[pl.ds(0, flat)]],
                                  mrb, sem2).wait()

            def maing(i, _):
                ds = pl.ds(i * 16, 16)
                tt = t3b[ds] * 2.0
                e = jnp.exp(-tt)
                sv = _sqrt(jnp.maximum(1.0 - e * e, EPS))
                pf = (1.0 - e) / (1.0 + e)
                bv = 2.0 * tt + 8.0 * e / (e + 1.0) - 4.0
                sr = VEL * _sqrt(jnp.maximum(bv, EPS))
                f0 = f0b[ds]
                v0 = v0b[ds]
                evc = VEL * (evb[ds] - mvb[ds])
                vt = e * v0 + sv * evc
                erc = erb[ds] - mrb[ds]
                mu = pf * (vt + v0)
                rt = _wrap(mu + sr * erc)
                fraw = _wrap(_wrap(f0) + rt)
                vtb[ds] = vt
                evcb[ds] = evc
                ercb[ds] = erc
                rtb[ds] = rt
                frawb[ds] = fraw
                return 0
            lax.fori_loop(0, flat // 16, maing, 0)

            pltpu.sync_copy(frawb, ftable.at[slb.at[pl.ds(0, flat)]],
                            add=True)
            pltpu.sync_copy(vtb, vt_out.at[pl.ds(f0o, flat)])
            pltpu.sync_copy(evcb, evc_out.at[pl.ds(f0o, flat)])
            pltpu.sync_copy(ercb, erc_out.at[pl.ds(f0o, flat)])
            pltpu.sync_copy(rtb, rt_out.at[pl.ds(f0o, flat)])
            pltpu.sync_copy(frawb, fraw_out.at[pl.ds(f0o, flat)])
            return 0
        lax.fori_loop(0, nt, tile, 0)
        plsc.subcore_barrier()
        _export_table(ftable, fp_out)

    def _k2b_body(p0, p1, c0, c1, o):
        x = p0[0] + p1[0]
        cnt = jnp.maximum(c0[0][:, 6:7] + c1[0][:, 6:7], 1.0)
        col = lax.broadcasted_iota(i32, (bs, 8), 1)
        o[...] = jnp.where(col < 3, x / cnt, 0.0)

    k2b = pl.pallas_call(
        _k2b_body,
        out_shape=jax.ShapeDtypeStruct((s, 8), f32),
        grid=(s // bs,),
        in_specs=[pl.BlockSpec((1, bs, 8), lambda i: (0, i, 0)),
                  pl.BlockSpec((1, bs, 8), lambda i: (1, i, 0)),
                  pl.BlockSpec((1, bs, 8), lambda i: (0, i, 0)),
                  pl.BlockSpec((1, bs, 8), lambda i: (1, i, 0))],
        out_specs=pl.BlockSpec((bs, 8), lambda i: (i, 0)),
        interpret=interpret,
    )

    @functools.partial(
        pl.kernel,
        out_type=jax.ShapeDtypeStruct((3 * n,), f32),
        mesh=mesh, interpret=interpret, compiler_params=cparams,
        scratch_types=[
            pltpu.VMEM((flat,), f32),
            pltpu.VMEM((flat,), f32),
            pltpu.VMEM((flat,), i32),
            pltpu.VMEM((flat,), f32),
            pltpu.SemaphoreType.DMA,
        ],
    )
    def k3(fraw_hbm, sl_hbm, fm_hbm, out_hbm, frawb, ftb, slb, fmb, sem):
        w = _wid()

        def tile(k, _):
            row0 = w * chunk + k * t_tile
            f0o = pl.multiple_of(row0 * 3, 8)
            pltpu.sync_copy(sl_hbm.at[pl.ds(f0o, flat)], slb)
            pltpu.async_copy(fm_hbm.at[slb.at[pl.ds(0, flat)]], fmb, sem)
            pltpu.sync_copy(fraw_hbm.at[pl.ds(f0o, flat)], frawb)
            pltpu.make_async_copy(fm_hbm.at[slb.at[pl.ds(0, flat)]],
                                  fmb, sem).wait()

            def maing(i, _):
                ds = pl.ds(i * 16, 16)
                ftb[ds] = frawb[ds] - fmb[ds]
                return 0
            lax.fori_loop(0, flat // 16, maing, 0)

            pltpu.sync_copy(ftb, out_hbm.at[pl.ds(f0o, flat)])
            return 0
        lax.fori_loop(0, nt, tile, 0)

    def run(t, f0, index, v0, epsilon_v, epsilon_r):
        idx2 = index.astype(i32).reshape(nr, L)
        t2 = t.reshape(nr, L)
        sl, sle, slc, t3 = tc0(idx2, t2)
        slf = sl.reshape(-1)
        slef = sle.reshape(-1)
        slcf = slc.reshape(-1)
        t3f = t3.reshape(-1)
        f0f = f0.reshape(-1)
        v0f = v0.reshape(-1)
        evf = epsilon_v.reshape(-1)
        erf = epsilon_r.reshape(-1)
        z = jnp.zeros((s8,), f32)
        part = k1(evf, erf, slf, slef, slcf, z)
        mvt, mrt = k1b(part.reshape(NC, s, 8), part.reshape(NC, s, 8))
        vtf, evcf, ercf, rtf, frawf, fpart = k2(
            t3f, f0f, v0f, evf, erf, slf,
            mvt.reshape(-1), mrt.reshape(-1), z)
        fmt = k2b(fpart.reshape(NC, s, 8), fpart.reshape(NC, s, 8),
                  part.reshape(NC, s, 8), part.reshape(NC, s, 8))
        ftf = k3(frawf, slf, fmt.reshape(-1))
        sh = (n, 3)
        return (ftf.reshape(sh), vtf.reshape(sh), evcf.reshape(sh),
                ercf.reshape(sh), rtf.reshape(sh))

    return run


_RUN = None


def kernel(t, f0, index, v0, epsilon_v, epsilon_r):
    global _RUN
    if _RUN is None:
        _RUN = _make(N, S, 2000)
    return _RUN(t, f0, index, v0, epsilon_v, epsilon_r)


# speedup vs baseline: 1.5023x; 1.0002x over previous
import functools
import math

import jax
import jax.numpy as jnp
from jax import lax
from jax.experimental import pallas as pl
from jax.experimental.pallas import tpu as pltpu
from jax.experimental.pallas import tpu_sc as plsc

N = 3200000
S = 100000
EPS = 1e-05
VEL = 1.0 / (2.0 * math.pi)
NC = 2
NS = 16
NW = NC * NS

f32 = jnp.float32
i32 = jnp.int32


def _sqrt(x):
    i = lax.bitcast_convert_type(x, i32)
    y = lax.bitcast_convert_type((i >> 1) + 0x1FBD1DF5, f32)
    y = 0.5 * (y + x / y)
    y = 0.5 * (y + x / y)
    y = 0.5 * (y + x / y)
    return y


def _wrap(x):
    m = lax.rem(x + 0.5, 1.0)
    m = jnp.where(m < 0.0, m + 1.0, m)
    return m - 0.5


def _make(n, s, t_tile, interpret=False):
    chunk = n // NW
    nt = chunk // t_tile
    flat = 3 * t_tile
    s8 = s * 8
    eb = s8 // NS
    assert n == NW * chunk and chunk == nt * t_tile
    assert t_tile % 16 == 0 and s8 % NS == 0 and eb % 8 == 0
    mesh = plsc.VectorSubcoreMesh(
        core_axis_name="c", subcore_axis_name="s",
        num_cores=NC, num_subcores=NS)
    cparams = pltpu.CompilerParams(use_tc_tiling_on_sc=False)

    def _wid():
        return lax.axis_index("s") * NC + lax.axis_index("c")

    def _zero_table(z_hbm, table):
        sid = lax.axis_index("s")
        pltpu.sync_copy(z_hbm.at[pl.ds(pl.multiple_of(sid * eb, 8), eb)],
                        table.at[pl.ds(pl.multiple_of(sid * eb, 8), eb)])

    def _export_table(table, out):
        sid = lax.axis_index("s")
        cid = lax.axis_index("c")
        pltpu.sync_copy(table.at[pl.ds(pl.multiple_of(sid * eb, 8), eb)],
                        out.at[cid, pl.ds(pl.multiple_of(sid * eb, 8), eb)])

    L = 128
    rb = 200
    nr = n // L
    assert n % L == 0 and nr % rb == 0

    def _tc0_body(ix, tt, sl_o, slc_o, t3_o):
        base = ix[...] * 8
        rep = jnp.reshape(jnp.broadcast_to(base[..., None], (rb, L, 3)),
                          (rb, L * 3))
        c3 = lax.broadcasted_iota(i32, (rb, L * 3), 1) % 3
        sl_o[...] = rep + c3
        slc_o[...] = base + 6
        t3_o[...] = jnp.reshape(
            jnp.broadcast_to(tt[...][..., None], (rb, L, 3)), (rb, L * 3))

    tc0 = pl.pallas_call(
        _tc0_body,
        out_shape=(jax.ShapeDtypeStruct((nr, L * 3), i32),
                   jax.ShapeDtypeStruct((nr, L), i32),
                   jax.ShapeDtypeStruct((nr, L * 3), f32)),
        grid=(nr // rb,),
        in_specs=[pl.BlockSpec((rb, L), lambda i: (i, 0)),
                  pl.BlockSpec((rb, L), lambda i: (i, 0))],
        out_specs=(pl.BlockSpec((rb, L * 3), lambda i: (i, 0)),
                   pl.BlockSpec((rb, L), lambda i: (i, 0)),
                   pl.BlockSpec((rb, L * 3), lambda i: (i, 0))),
        interpret=interpret,
    )

    @functools.partial(
        pl.kernel,
        out_type=jax.ShapeDtypeStruct((NC, s8), f32),
        mesh=mesh, interpret=interpret, compiler_params=cparams,
        scratch_types=[
            pltpu.VMEM((flat,), f32),
            pltpu.VMEM((flat,), f32),
            pltpu.VMEM((flat,), i32),
            pltpu.VMEM((flat,), i32),
            pltpu.VMEM((t_tile,), i32),
            pltpu.VMEM((t_tile,), f32),
            pltpu.VMEM_SHARED((s8,), f32),
        ],
    )
    def k1(ev_hbm, er_hbm, sl_hbm, slc_hbm, z_hbm, out_hbm,
           evb, erb, slb, sleb, slcb, ones, table):
        w = _wid()
        _zero_table(z_hbm, table)

        def oinit(i, _):
            ones[pl.ds(i * 16, 16)] = jnp.ones((16,), f32)
            return 0
        lax.fori_loop(0, t_tile // 16, oinit, 0)
        plsc.subcore_barrier()

        def tile(k, _):
            row0 = w * chunk + k * t_tile
            f0 = pl.multiple_of(row0 * 3, 8)
            r0 = pl.multiple_of(row0, 8)
            pltpu.sync_copy(ev_hbm.at[pl.ds(f0, flat)], evb)
            pltpu.sync_copy(er_hbm.at[pl.ds(f0, flat)], erb)
            pltpu.sync_copy(sl_hbm.at[pl.ds(f0, flat)], slb)
            pltpu.sync_copy(slc_hbm.at[pl.ds(r0, t_tile)], slcb)

            def sle_fill(i, _):
                ds = pl.ds(i * 16, 16)
                sleb[ds] = slb[ds] + 3
                return 0
            lax.fori_loop(0, flat // 16, sle_fill, 0)
            pltpu.sync_copy(evb, table.at[slb.at[pl.ds(0, flat)]], add=True)
            pltpu.sync_copy(erb, table.at[sleb.at[pl.ds(0, flat)]], add=True)
            pltpu.sync_copy(ones, table.at[slcb.at[pl.ds(0, t_tile)]],
                            add=True)
            return 0
        lax.fori_loop(0, nt, tile, 0)
        plsc.subcore_barrier()
        _export_table(table, out_hbm)

    bs = 2000
    assert s % bs == 0

    def _k1b_body(p0, p1, mv_o, mr_o):
        x = p0[0] + p1[0]
        cnt = jnp.maximum(x[:, 6:7], 1.0)
        col = lax.broadcasted_iota(i32, (bs, 8), 1)
        mv_o[...] = jnp.where(col < 3, x / cnt, 0.0)
        mr_o[...] = jnp.concatenate(
            [x[:, 3:6] / cnt, jnp.zeros((bs, 5), f32)], axis=1)

    k1b = pl.pallas_call(
        _k1b_body,
        out_shape=(jax.ShapeDtypeStruct((s, 8), f32),
                   jax.ShapeDtypeStruct((s, 8), f32)),
        grid=(s // bs,),
        in_specs=[pl.BlockSpec((1, bs, 8), lambda i: (0, i, 0)),
                  pl.BlockSpec((1, bs, 8), lambda i: (1, i, 0))],
        out_specs=(pl.BlockSpec((bs, 8), lambda i: (i, 0)),
                   pl.BlockSpec((bs, 8), lambda i: (i, 0))),
        interpret=interpret,
    )

    of = jax.ShapeDtypeStruct((3 * n,), f32)

    @functools.partial(
        pl.kernel,
        out_type=(of, of, of, of, of,
                  jax.ShapeDtypeStruct((NC, s8), f32)),
        mesh=mesh, interpret=interpret, compiler_params=cparams,
        scratch_types=[
            pltpu.VMEM((flat,), f32),
            pltpu.VMEM((flat,), f32),
            pltpu.VMEM((flat,), f32),
            pltpu.VMEM((flat,), f32),
            pltpu.VMEM((flat,), f32),
            pltpu.VMEM((flat,), i32),
            pltpu.VMEM((flat,), f32),
            pltpu.VMEM((flat,), f32),
            pltpu.VMEM((flat,), f32),
            pltpu.VMEM((flat,), f32),
            pltpu.VMEM((flat,), f32),
            pltpu.VMEM((flat,), f32),
            pltpu.VMEM((flat,), f32),
            pltpu.VMEM_SHARED((s8,), f32),
            pltpu.SemaphoreType.DMA,
            pltpu.SemaphoreType.DMA,
        ],
    )
    def k2(t3_hbm, f0_hbm, v0_hbm, ev_hbm, er_hbm, sl_hbm, mv_hbm, mr_hbm,
           z_hbm,
           vt_out, evc_out, erc_out, rt_out, fraw_out, fp_out,
           t3b, f0b, v0b, evb, erb, slb, mvb, mrb,
           vtb, evcb, ercb, rtb, frawb, ftable, sem1, sem2):
        w = _wid()
        _zero_table(z_hbm, ftable)
        plsc.subcore_barrier()

        def tile(k, _):
            row0 = w * chunk + k * t_tile
            f0o = pl.multiple_of(row0 * 3, 8)
            pltpu.sync_copy(sl_hbm.at[pl.ds(f0o, flat)], slb)
            pltpu.async_copy(mv_hbm.at[slb.at[pl.ds(0, flat)]], mvb, sem1)
            pltpu.async_copy(mr_hbm.at[slb.at[pl.ds(0, flat)]], mrb, sem2)
            pltpu.sync_copy(t3_hbm.at[pl.ds(f0o, flat)], t3b)
            pltpu.sync_copy(f0_hbm.at[pl.ds(f0o, flat)], f0b)
            pltpu.sync_copy(v0_hbm.at[pl.ds(f0o, flat)], v0b)
            pltpu.sync_copy(ev_hbm.at[pl.ds(f0o, flat)], evb)
            pltpu.sync_copy(er_hbm.at[pl.ds(f0o, flat)], erb)
            pltpu.make_async_copy(mv_hbm.at[slb.at[pl.ds(0, flat)]],
                                  mvb, sem1).wait()
            pltpu.make_async_copy(mr_hbm.at[slb.at[pl.ds(0, flat)]],
                                  mrb, sem2).wait()

            def maing(i, _):
                ds = pl.ds(i * 16, 16)
                tt = t3b[ds] * 2.0
                e = jnp.exp(-tt)
                sv = _sqrt(jnp.maximum(1.0 - e * e, EPS))
                pf = (1.0 - e) / (1.0 + e)
                bv = 2.0 * tt + 8.0 * e / (e + 1.0) - 4.0
                sr = VEL * _sqrt(jnp.maximum(bv, EPS))
                f0 = f0b[ds]
                v0 = v0b[ds]
                evc = VEL * (evb[ds] - mvb[ds])
                vt = e * v0 + sv * evc
                erc = erb[ds] - mrb[ds]
                mu = pf * (vt + v0)
                rt = _wrap(mu + sr * erc)
                fraw = _wrap(_wrap(f0) + rt)
                vtb[ds] = vt
                evcb[ds] = evc
                ercb[ds] = erc
                rtb[ds] = rt
                frawb[ds] = fraw
                return 0
            lax.fori_loop(0, flat // 16, maing, 0)

            pltpu.sync_copy(frawb, ftable.at[slb.at[pl.ds(0, flat)]],
                            add=True)
            pltpu.sync_copy(vtb, vt_out.at[pl.ds(f0o, flat)])
            pltpu.sync_copy(evcb, evc_out.at[pl.ds(f0o, flat)])
            pltpu.sync_copy(ercb, erc_out.at[pl.ds(f0o, flat)])
            pltpu.sync_copy(rtb, rt_out.at[pl.ds(f0o, flat)])
            pltpu.sync_copy(frawb, fraw_out.at[pl.ds(f0o, flat)])
            return 0
        lax.fori_loop(0, nt, tile, 0)
        plsc.subcore_barrier()
        _export_table(ftable, fp_out)

    def _k2b_body(p0, p1, c0, c1, o):
        x = p0[0] + p1[0]
        cnt = jnp.maximum(c0[0][:, 6:7] + c1[0][:, 6:7], 1.0)
        col = lax.broadcasted_iota(i32, (bs, 8), 1)
        o[...] = jnp.where(col < 3, x / cnt, 0.0)

    k2b = pl.pallas_call(
        _k2b_body,
        out_shape=jax.ShapeDtypeStruct((s, 8), f32),
        grid=(s // bs,),
        in_specs=[pl.BlockSpec((1, bs, 8), lambda i: (0, i, 0)),
                  pl.BlockSpec((1, bs, 8), lambda i: (1, i, 0)),
                  pl.BlockSpec((1, bs, 8), lambda i: (0, i, 0)),
                  pl.BlockSpec((1, bs, 8), lambda i: (1, i, 0))],
        out_specs=pl.BlockSpec((bs, 8), lambda i: (i, 0)),
        interpret=interpret,
    )

    @functools.partial(
        pl.kernel,
        out_type=jax.ShapeDtypeStruct((3 * n,), f32),
        mesh=mesh, interpret=interpret, compiler_params=cparams,
        scratch_types=[
            pltpu.VMEM((flat,), f32),
            pltpu.VMEM((flat,), f32),
            pltpu.VMEM((flat,), i32),
            pltpu.VMEM((flat,), f32),
            pltpu.SemaphoreType.DMA,
        ],
    )
    def k3(fraw_hbm, sl_hbm, fm_hbm, out_hbm, frawb, ftb, slb, fmb, sem):
        w = _wid()

        def tile(k, _):
            row0 = w * chunk + k * t_tile
            f0o = pl.multiple_of(row0 * 3, 8)
            pltpu.sync_copy(sl_hbm.at[pl.ds(f0o, flat)], slb)
            pltpu.async_copy(fm_hbm.at[slb.at[pl.ds(0, flat)]], fmb, sem)
            pltpu.sync_copy(fraw_hbm.at[pl.ds(f0o, flat)], frawb)
            pltpu.make_async_copy(fm_hbm.at[slb.at[pl.ds(0, flat)]],
                                  fmb, sem).wait()

            def maing(i, _):
                ds = pl.ds(i * 16, 16)
                ftb[ds] = frawb[ds] - fmb[ds]
                return 0
            lax.fori_loop(0, flat // 16, maing, 0)

            pltpu.sync_copy(ftb, out_hbm.at[pl.ds(f0o, flat)])
            return 0
        lax.fori_loop(0, nt, tile, 0)

    def run(t, f0, index, v0, epsilon_v, epsilon_r):
        idx2 = index.astype(i32).reshape(nr, L)
        t2 = t.reshape(nr, L)
        sl, slc, t3 = tc0(idx2, t2)
        slf = sl.reshape(-1)
        slcf = slc.reshape(-1)
        t3f = t3.reshape(-1)
        f0f = f0.reshape(-1)
        v0f = v0.reshape(-1)
        evf = epsilon_v.reshape(-1)
        erf = epsilon_r.reshape(-1)
        z = jnp.zeros((s8,), f32)
        part = k1(evf, erf, slf, slcf, z)
        mvt, mrt = k1b(part.reshape(NC, s, 8), part.reshape(NC, s, 8))
        vtf, evcf, ercf, rtf, frawf, fpart = k2(
            t3f, f0f, v0f, evf, erf, slf,
            mvt.reshape(-1), mrt.reshape(-1), z)
        fmt = k2b(fpart.reshape(NC, s, 8), fpart.reshape(NC, s, 8),
                  part.reshape(NC, s, 8), part.reshape(NC, s, 8))
        ftf = k3(frawf, slf, fmt.reshape(-1))
        sh = (n, 3)
        return (ftf.reshape(sh), vtf.reshape(sh), evcf.reshape(sh),
                ercf.reshape(sh), rtf.reshape(sh))

    return run


_RUN = None


def kernel(t, f0, index, v0, epsilon_v, epsilon_r):
    global _RUN
    if _RUN is None:
        _RUN = _make(N, S, 2000)
    return _RUN(t, f0, index, v0, epsilon_v, epsilon_r)
